# baseline (speedup 1.0000x reference)
import jax
import jax.numpy as jnp
from jax.experimental import pallas as pl
from jax.experimental.pallas import tpu as pltpu

N = 4096
F = 16
BM = 256
NB = N // BM


def _fused_kernel(adj_ref, w_ref, out_ref, z_scr, zt_scr):
    p = pl.program_id(0)
    i = pl.program_id(1)

    @pl.when(p == 0)
    def _encode():
        zi = jnp.dot(adj_ref[...], w_ref[...],
                     preferred_element_type=jnp.float32)
        z_scr[pl.ds(i * BM, BM), :] = zi
        zt_scr[:, pl.ds(i * BM, BM)] = zi.T

    @pl.when(p == 1)
    def _decode():
        out_ref[...] = jnp.dot(z_scr[pl.ds(i * BM, BM), :], zt_scr[...],
                               preferred_element_type=jnp.float32)


@jax.jit
def kernel(adj, W):
    out = pl.pallas_call(
        _fused_kernel,
        grid=(2, NB),
        in_specs=[
            pl.BlockSpec((BM, N), lambda p, i: (jnp.where(p == 0, i, NB - 1), 0)),
            pl.BlockSpec((N, F), lambda p, i: (0, 0)),
        ],
        out_specs=pl.BlockSpec((BM, N), lambda p, i: (jnp.where(p == 0, 0, i), 0)),
        out_shape=jax.ShapeDtypeStruct((N, N), jnp.float32),
        scratch_shapes=[
            pltpu.VMEM((N, F), jnp.float32),
            pltpu.VMEM((F, N), jnp.float32),
        ],
    )(adj, W)
    return out


# speedup vs baseline: 1.0493x; 1.0493x over previous
import jax
import jax.numpy as jnp
from jax.experimental import pallas as pl
from jax.experimental.pallas import tpu as pltpu

N = 4096
F = 16
BME = 512
NBE = N // BME
BMD = 1024
NBD = N // BMD


def _fused_kernel(adj_ref, w_ref, out_ref, z_scr, zt_scr):
    i = pl.program_id(0)

    @pl.when(i < NBE)
    def _encode():
        zi = jnp.dot(adj_ref[...], w_ref[...],
                     preferred_element_type=jnp.float32)
        z_scr[pl.ds(i * BME, BME), :] = zi
        zt_scr[:, pl.ds(i * BME, BME)] = zi.T

    @pl.when(i >= NBE)
    def _decode():
        j = i - NBE
        out_ref[...] = jnp.dot(z_scr[pl.ds(j * BMD, BMD), :], zt_scr[...],
                               preferred_element_type=jnp.float32)


@jax.jit
def kernel(adj, W):
    out = pl.pallas_call(
        _fused_kernel,
        grid=(NBE + NBD,),
        in_specs=[
            pl.BlockSpec((BME, N), lambda i: (jnp.where(i < NBE, i, NBE - 1), 0)),
            pl.BlockSpec((N, F), lambda i: (0, 0)),
        ],
        out_specs=pl.BlockSpec((BMD, N),
                               lambda i: (jnp.where(i < NBE, 0, i - NBE), 0)),
        out_shape=jax.ShapeDtypeStruct((N, N), jnp.float32),
        scratch_shapes=[
            pltpu.VMEM((N, F), jnp.float32),
            pltpu.VMEM((F, N), jnp.float32),
        ],
    )(adj, W)
    return out


# speedup vs baseline: 1.0897x; 1.0386x over previous
import jax
import jax.numpy as jnp
from jax.experimental import pallas as pl
from jax.experimental.pallas import tpu as pltpu

N = 4096
F = 16
BM = 512
NB = N // BM


def _fused_kernel(adj_ref, w_ref, out_ref, z_scr, zt_scr):
    p = pl.program_id(0)
    i = pl.program_id(1)

    @pl.when(p == 0)
    def _encode():
        zi = jnp.dot(adj_ref[...], w_ref[...],
                     preferred_element_type=jnp.float32)
        z_scr[pl.ds(i * BM, BM), :] = zi
        zt_scr[:, pl.ds(i * BM, BM)] = zi.T

    @pl.when(p == 1)
    def _decode():
        out_ref[...] = jnp.dot(z_scr[pl.ds(i * BM, BM), :], zt_scr[...],
                               preferred_element_type=jnp.float32)


@jax.jit
def kernel(adj, W):
    out = pl.pallas_call(
        _fused_kernel,
        grid=(2, NB),
        in_specs=[
            pl.BlockSpec((BM, N), lambda p, i: (jnp.where(p == 0, i, NB - 1), 0)),
            pl.BlockSpec((N, F), lambda p, i: (0, 0)),
        ],
        out_specs=pl.BlockSpec((BM, N), lambda p, i: (jnp.where(p == 0, 0, i), 0)),
        out_shape=jax.ShapeDtypeStruct((N, N), jnp.float32),
        scratch_shapes=[
            pltpu.VMEM((N, F), jnp.float32),
            pltpu.VMEM((F, N), jnp.float32),
        ],
    )(adj, W)
    return out
